# baseline (speedup 1.0000x reference)
import functools

import jax
import jax.numpy as jnp
from jax import lax
from jax.experimental import pallas as pl
from jax.experimental.pallas import tpu as pltpu
from jax.experimental.pallas import tpu_sc as plsc

N = 10000
E = 160000
C = 256
EPS = 1e-6

NC = 2
NS = 16
EPT = E // NS
WE = 2000
NW = EPT // WE
WIT = WE // 16
CHUNK = 1024
NCHUNK = 10
NPAD = CHUNK * NCHUNK
ACC_B = 2 * CHUNK + 16
G = 64
BUF = ((EPT + G - 1) // G) * G
ZROWS = 32
WB = CHUNK // NS
XB = 2 * N
OB = 2 * NPAD


def _sc_body(x_hbm, src_hbm, dst_hbm, out_hbm,
             src_w, dst_w, srcbuf, ldstbuf, gidx, sidx, rows_v, zrow_v,
             acc_sh, sem, zsem):
    core = lax.axis_index("c")
    sid = lax.axis_index("s")

    zvec = jnp.zeros((16,), jnp.float32)
    for r in range(ZROWS):
        def _zb(j, carry, r=r):
            zrow_v[r, pl.ds(j * 16, 16)] = zvec
            return carry
        lax.fori_loop(0, 128 // 16, _zb, 0)

    lane = jnp.arange(16, dtype=jnp.int32)
    pad_src = lane + sid * 16
    pad_ldst = CHUNK + (lane & 7)

    for p in range(NCHUNK // NC):
        chunk = core * (NCHUNK // NC) + p
        lo = chunk * CHUNK

        zcps = []
        for k in range(3):
            srow = k * ACC_B + 2 * sid * WB
            for t in range((2 * WB) // ZROWS):
                zcps.append(pltpu.async_copy(
                    zrow_v, acc_sh.at[pl.ds(srow + t * ZROWS, ZROWS)],
                    zsem))
            zcps.append(pltpu.async_copy(
                zrow_v.at[pl.ds(0, 1)],
                acc_sh.at[pl.ds(k * ACC_B + 2 * CHUNK + sid, 1)], zsem))
        for cp in zcps:
            cp.wait()
        plsc.subcore_barrier()

        cnt = jnp.int32(0)
        for w in range(NW):
            base_e = sid * EPT + w * WE
            cp1 = pltpu.async_copy(src_hbm.at[pl.ds(base_e, WE)], src_w,
                                   zsem)
            cp2 = pltpu.async_copy(dst_hbm.at[pl.ds(base_e, WE)], dst_w,
                                   zsem)
            cp1.wait()
            cp2.wait()

            def _cp(i, cnt):
                s = src_w[pl.ds(i * 16, 16)]
                d = dst_w[pl.ds(i * 16, 16)]
                m = (d >= lo) & (d < lo + CHUNK)
                pos = cnt + plsc.cumsum(
                    jnp.where(m, 1, 0).astype(jnp.int32)) - 1
                plsc.store_scatter(srcbuf, [pos], s, mask=m)
                plsc.store_scatter(ldstbuf, [pos], d - lo, mask=m)
                return jnp.max(pos) + 1
            cnt = lax.fori_loop(0, WIT, _cp, cnt)

        for j in range(G // 16):
            srcbuf[pl.ds(cnt + j * 16, 16)] = pad_src
            ldstbuf[pl.ds(cnt + j * 16, 16)] = pad_ldst

        nb = (cnt + G - 1) // G
        def _gs(b, carry):
            for j in range(G // 16):
                sv = srcbuf[pl.ds(b * G + j * 16, 16)] * 2
                dv = ldstbuf[pl.ds(b * G + j * 16, 16)] * 2
                for k in range(6):
                    gidx[k, pl.ds(j * 16, 16)] = sv + ((k // 2) * XB
                                                       + (k % 2))
                    sidx[k, pl.ds(j * 16, 16)] = dv + ((k // 2) * ACC_B
                                                       + (k % 2))
            cps = [pltpu.async_copy(x_hbm.at[gidx.at[k]], rows_v.at[k], sem)
                   for k in range(6)]
            for cp in cps:
                cp.wait()
            for k in range(6):
                pltpu.sync_copy(rows_v.at[k], acc_sh.at[sidx.at[k]],
                                add=True)
            return carry
        lax.fori_loop(0, nb, _gs, 0)
        plsc.subcore_barrier()

        wcps = []
        for k in range(3):
            wrow = 2 * sid * WB
            wcps.append(pltpu.async_copy(
                acc_sh.at[pl.ds(k * ACC_B + wrow, 2 * WB)],
                out_hbm.at[pl.ds(k * OB + 2 * lo + wrow, 2 * WB)], zsem))
        for cp in wcps:
            cp.wait()


@functools.lru_cache(maxsize=1)
def _get_seg_sum():
    return pl.kernel(
        _sc_body,
        out_type=jax.ShapeDtypeStruct((3 * OB, 128), jnp.float32),
        mesh=plsc.VectorSubcoreMesh(core_axis_name="c", subcore_axis_name="s",
                                    num_cores=NC, num_subcores=NS),
        compiler_params=pltpu.CompilerParams(needs_layout_passes=False),
        scratch_types=[
            pltpu.VMEM((WE,), jnp.int32),
            pltpu.VMEM((WE,), jnp.int32),
            pltpu.VMEM((BUF + G,), jnp.int32),
            pltpu.VMEM((BUF + G,), jnp.int32),
            pltpu.VMEM((6, G), jnp.int32),
            pltpu.VMEM((6, G), jnp.int32),
            pltpu.VMEM((6, G, 128), jnp.float32),
            pltpu.VMEM((ZROWS, 128), jnp.float32),
            pltpu.VMEM_SHARED((3 * ACC_B, 128), jnp.float32),
            pltpu.SemaphoreType.DMA,
            pltpu.SemaphoreType.DMA,
        ],
    )


BN = 400


def _tc_body(x_ref, p_ref, wid_ref, wpool_ref, wdir_ref, bias_ref, out_ref):
    xb = x_ref[...]
    x2 = xb.reshape(3 * BN, C)
    p2 = p_ref[...].reshape(3 * BN, C)
    emb = (jnp.dot(x2.astype(jnp.bfloat16), wid_ref[...],
                   preferred_element_type=jnp.float32)
           + jnp.dot(p2.astype(jnp.bfloat16), wpool_ref[...],
                     preferred_element_type=jnp.float32)
           + bias_ref[...])
    d = jnp.dot(emb.astype(jnp.bfloat16), wdir_ref[...],
                preferred_element_type=jnp.float32)
    e3 = emb.reshape(3, BN, C)
    d3 = d.reshape(3, BN, C)
    dot = jnp.sum(e3 * d3, axis=0, keepdims=True)
    dn = jnp.sum(d3 * d3, axis=0, keepdims=True)
    coef = jnp.where(dot < 0.0, dot / (dn + EPS), 0.0)
    out_ref[...] = e3 - coef * d3 + xb


_tc_call = pl.pallas_call(
    _tc_body,
    grid=(N // BN,),
    in_specs=[
        pl.BlockSpec((3, BN, C), lambda i: (0, i, 0)),
        pl.BlockSpec((3, BN, C), lambda i: (0, i, 0)),
        pl.BlockSpec((C, C), lambda i: (0, 0)),
        pl.BlockSpec((C, C), lambda i: (0, 0)),
        pl.BlockSpec((C, C), lambda i: (0, 0)),
        pl.BlockSpec((1, C), lambda i: (0, 0)),
    ],
    out_specs=pl.BlockSpec((3, BN, C), lambda i: (0, i, 0)),
    out_shape=jax.ShapeDtypeStruct((3, N, C), jnp.float32),
)


def kernel(x, edges, W_id, b_id, W_pool, b_pool, W_dir):
    xt = jnp.transpose(x, (1, 0, 2))
    x2d = xt.reshape(3 * XB, 128)
    src = edges[0]
    dst = edges[1]
    pooled = _get_seg_sum()(x2d, src, dst)
    pooled_t = pooled.reshape(3, NPAD, C)
    bias = (b_id + b_pool).reshape(1, C)
    out_t = _tc_call(xt, pooled_t,
                     W_id.T.astype(jnp.bfloat16),
                     W_pool.T.astype(jnp.bfloat16),
                     W_dir.T.astype(jnp.bfloat16), bias)
    out = jnp.transpose(out_t, (1, 0, 2))
    return (out, edges)


# speedup vs baseline: 1.1711x; 1.1711x over previous
import functools

import jax
import jax.numpy as jnp
from jax import lax
from jax.experimental import pallas as pl
from jax.experimental.pallas import tpu as pltpu
from jax.experimental.pallas import tpu_sc as plsc

N = 10000
E = 160000
C = 256
EPS = 1e-6

NC = 2
NS = 16
EPT = E // NS
WE = 2000
NW = EPT // WE
WIT = WE // 16
CHUNK = 1024
NCHUNK = 10
NPAD = CHUNK * NCHUNK
ACC_B = 2 * CHUNK + 16
G = 64
BUF = ((EPT + G - 1) // G) * G
ZROWS = 32
WB = CHUNK // NS
XB = 2 * N
OB = 2 * NPAD


def _sc_body(x_hbm, src_hbm, dst_hbm, out_hbm,
             src_w, dst_w, srcbuf, ldstbuf, gidx, sidx, rows_v, zrow_v,
             acc_sh, sem, zsem):
    core = lax.axis_index("c")
    sid = lax.axis_index("s")

    zvec = jnp.zeros((16,), jnp.float32)
    for r in range(ZROWS):
        def _zb(j, carry, r=r):
            zrow_v[r, pl.ds(j * 16, 16)] = zvec
            return carry
        lax.fori_loop(0, 128 // 16, _zb, 0)

    lane = jnp.arange(16, dtype=jnp.int32)
    pad_src = lane + sid * 16
    pad_ldst = CHUNK + (lane & 7)

    for p in range(NCHUNK // NC):
        chunk = core * (NCHUNK // NC) + p
        lo = chunk * CHUNK

        zcps = []
        for k in range(3):
            srow = k * ACC_B + 2 * sid * WB
            for t in range((2 * WB) // ZROWS):
                zcps.append(pltpu.async_copy(
                    zrow_v, acc_sh.at[pl.ds(srow + t * ZROWS, ZROWS)],
                    zsem))
            zcps.append(pltpu.async_copy(
                zrow_v.at[pl.ds(0, 1)],
                acc_sh.at[pl.ds(k * ACC_B + 2 * CHUNK + sid, 1)], zsem))
        for cp in zcps:
            cp.wait()
        plsc.subcore_barrier()

        cnt = jnp.int32(0)
        for w in range(NW):
            base_e = sid * EPT + w * WE
            cp1 = pltpu.async_copy(src_hbm.at[pl.ds(base_e, WE)], src_w,
                                   zsem)
            cp2 = pltpu.async_copy(dst_hbm.at[pl.ds(base_e, WE)], dst_w,
                                   zsem)
            cp1.wait()
            cp2.wait()

            def _cp(i, cnt):
                s = src_w[pl.ds(i * 16, 16)]
                d = dst_w[pl.ds(i * 16, 16)]
                m = (d >= lo) & (d < lo + CHUNK)
                pos = cnt + plsc.cumsum(
                    jnp.where(m, 1, 0).astype(jnp.int32)) - 1
                plsc.store_scatter(srcbuf, [pos], s, mask=m)
                plsc.store_scatter(ldstbuf, [pos], d - lo, mask=m)
                return jnp.max(pos) + 1
            cnt = lax.fori_loop(0, WIT, _cp, cnt)

        for j in range(G // 16):
            srcbuf[pl.ds(cnt + j * 16, 16)] = pad_src
            ldstbuf[pl.ds(cnt + j * 16, 16)] = pad_ldst

        nb = (cnt + G - 1) // G
        def _gs(b, carry):
            for j in range(G // 16):
                sv = srcbuf[pl.ds(b * G + j * 16, 16)] * 2
                dv = ldstbuf[pl.ds(b * G + j * 16, 16)] * 2
                for k in range(6):
                    gidx[k, pl.ds(j * 16, 16)] = sv + ((k // 2) * XB
                                                       + (k % 2))
                    sidx[k, pl.ds(j * 16, 16)] = dv + ((k // 2) * ACC_B
                                                       + (k % 2))
            cps = [pltpu.async_copy(x_hbm.at[gidx.at[k]], rows_v.at[k], sem)
                   for k in range(6)]
            scps = []
            for k in range(6):
                cps[k].wait()
                scps.append(pltpu.async_copy(
                    rows_v.at[k], acc_sh.at[sidx.at[k]], zsem, add=True))
            for cp in scps:
                cp.wait()
            return carry
        lax.fori_loop(0, nb, _gs, 0)
        plsc.subcore_barrier()

        wcps = []
        for k in range(3):
            wrow = 2 * sid * WB
            wcps.append(pltpu.async_copy(
                acc_sh.at[pl.ds(k * ACC_B + wrow, 2 * WB)],
                out_hbm.at[pl.ds(k * OB + 2 * lo + wrow, 2 * WB)], zsem))
        for cp in wcps:
            cp.wait()


@functools.lru_cache(maxsize=1)
def _get_seg_sum():
    return pl.kernel(
        _sc_body,
        out_type=jax.ShapeDtypeStruct((3 * OB, 128), jnp.float32),
        mesh=plsc.VectorSubcoreMesh(core_axis_name="c", subcore_axis_name="s",
                                    num_cores=NC, num_subcores=NS),
        compiler_params=pltpu.CompilerParams(needs_layout_passes=False),
        scratch_types=[
            pltpu.VMEM((WE,), jnp.int32),
            pltpu.VMEM((WE,), jnp.int32),
            pltpu.VMEM((BUF + G,), jnp.int32),
            pltpu.VMEM((BUF + G,), jnp.int32),
            pltpu.VMEM((6, G), jnp.int32),
            pltpu.VMEM((6, G), jnp.int32),
            pltpu.VMEM((6, G, 128), jnp.float32),
            pltpu.VMEM((ZROWS, 128), jnp.float32),
            pltpu.VMEM_SHARED((3 * ACC_B, 128), jnp.float32),
            pltpu.SemaphoreType.DMA,
            pltpu.SemaphoreType.DMA,
        ],
    )


BN = 400


def _tc_body(x_ref, p_ref, wid_ref, wpool_ref, wdir_ref, bias_ref, out_ref):
    xb = x_ref[...]
    x2 = xb.reshape(3 * BN, C)
    p2 = p_ref[...].reshape(3 * BN, C)
    emb = (jnp.dot(x2.astype(jnp.bfloat16), wid_ref[...],
                   preferred_element_type=jnp.float32)
           + jnp.dot(p2.astype(jnp.bfloat16), wpool_ref[...],
                     preferred_element_type=jnp.float32)
           + bias_ref[...])
    d = jnp.dot(emb.astype(jnp.bfloat16), wdir_ref[...],
                preferred_element_type=jnp.float32)
    e3 = emb.reshape(3, BN, C)
    d3 = d.reshape(3, BN, C)
    dot = jnp.sum(e3 * d3, axis=0, keepdims=True)
    dn = jnp.sum(d3 * d3, axis=0, keepdims=True)
    coef = jnp.where(dot < 0.0, dot / (dn + EPS), 0.0)
    out_ref[...] = e3 - coef * d3 + xb


_tc_call = pl.pallas_call(
    _tc_body,
    grid=(N // BN,),
    in_specs=[
        pl.BlockSpec((3, BN, C), lambda i: (0, i, 0)),
        pl.BlockSpec((3, BN, C), lambda i: (0, i, 0)),
        pl.BlockSpec((C, C), lambda i: (0, 0)),
        pl.BlockSpec((C, C), lambda i: (0, 0)),
        pl.BlockSpec((C, C), lambda i: (0, 0)),
        pl.BlockSpec((1, C), lambda i: (0, 0)),
    ],
    out_specs=pl.BlockSpec((3, BN, C), lambda i: (0, i, 0)),
    out_shape=jax.ShapeDtypeStruct((3, N, C), jnp.float32),
)


def kernel(x, edges, W_id, b_id, W_pool, b_pool, W_dir):
    xt = jnp.transpose(x, (1, 0, 2))
    x2d = xt.reshape(3 * XB, 128)
    src = edges[0]
    dst = edges[1]
    pooled = _get_seg_sum()(x2d, src, dst)
    pooled_t = pooled.reshape(3, NPAD, C)
    bias = (b_id + b_pool).reshape(1, C)
    out_t = _tc_call(xt, pooled_t,
                     W_id.T.astype(jnp.bfloat16),
                     W_pool.T.astype(jnp.bfloat16),
                     W_dir.T.astype(jnp.bfloat16), bias)
    out = jnp.transpose(out_t, (1, 0, 2))
    return (out, edges)


# speedup vs baseline: 1.2323x; 1.0523x over previous
import functools

import jax
import jax.numpy as jnp
from jax import lax
from jax.experimental import pallas as pl
from jax.experimental.pallas import tpu as pltpu
from jax.experimental.pallas import tpu_sc as plsc

N = 10000
E = 160000
C = 256
EPS = 1e-6

NC = 2
NS = 16
EPT = E // NS
WE = 2000
NW = EPT // WE
WIT = WE // 16
CHUNK = 1024
NCHUNK = 10
NPAD = CHUNK * NCHUNK
ACC_B = 2 * CHUNK + 16
G = 64
BUF = ((EPT + G - 1) // G) * G
ZROWS = 8
WB = CHUNK // NS
XB = 2 * N
OB = 2 * NPAD


def _sc_body(x_hbm, src_hbm, dst_hbm, out_hbm,
             src_w0, src_w1, dst_w0, dst_w1, srcbuf, ldstbuf, gidx, sidx,
             rows_v, zrow_v, acc_sh, sem, zsem, wsem, bsem):
    src_w = (src_w0, src_w1)
    dst_w = (dst_w0, dst_w1)
    core = lax.axis_index("c")
    sid = lax.axis_index("s")

    zvec = jnp.zeros((16,), jnp.float32)
    for r in range(ZROWS):
        def _zb(j, carry, r=r):
            zrow_v[r, pl.ds(j * 16, 16)] = zvec
            return carry
        lax.fori_loop(0, 128 // 16, _zb, 0)

    lane = jnp.arange(16, dtype=jnp.int32)
    pad_src = lane + sid * 16
    pad_ldst = CHUNK + (lane & 7)

    wb_prev = []
    for p in range(NCHUNK // NC):
        chunk = core * (NCHUNK // NC) + p
        lo = chunk * CHUNK

        def _win_load(w):
            base_e = sid * EPT + w * WE
            return [pltpu.async_copy(src_hbm.at[pl.ds(base_e, WE)],
                                     src_w[w % 2], wsem),
                    pltpu.async_copy(dst_hbm.at[pl.ds(base_e, WE)],
                                     dst_w[w % 2], wsem)]
        cnt = jnp.int32(0)
        pend = _win_load(0)
        for w in range(NW):
            cur, pend = pend, (_win_load(w + 1) if w + 1 < NW else [])
            for cp in cur:
                cp.wait()

            def _cp(i, cnt, w=w):
                s = src_w[w % 2][pl.ds(i * 16, 16)]
                d = dst_w[w % 2][pl.ds(i * 16, 16)]
                m = (d >= lo) & (d < lo + CHUNK)
                pos = cnt + plsc.cumsum(
                    jnp.where(m, 1, 0).astype(jnp.int32)) - 1
                plsc.store_scatter(srcbuf, [pos], s, mask=m)
                plsc.store_scatter(ldstbuf, [pos], d - lo, mask=m)
                return jnp.max(pos) + 1
            cnt = lax.fori_loop(0, WIT, _cp, cnt)

        for cp in wb_prev:
            cp.wait()
        zcps = []
        for k in range(3):
            srow = k * ACC_B + 2 * sid * WB
            for t in range((2 * WB) // ZROWS):
                zcps.append(pltpu.async_copy(
                    zrow_v, acc_sh.at[pl.ds(srow + t * ZROWS, ZROWS)],
                    zsem))
            zcps.append(pltpu.async_copy(
                zrow_v.at[pl.ds(0, 1)],
                acc_sh.at[pl.ds(k * ACC_B + 2 * CHUNK + sid, 1)], zsem))
        for cp in zcps:
            cp.wait()
        plsc.subcore_barrier()

        for j in range(G // 16):
            srcbuf[pl.ds(cnt + j * 16, 16)] = pad_src
            ldstbuf[pl.ds(cnt + j * 16, 16)] = pad_ldst

        nb = (cnt + G - 1) // G
        def _gs(b, carry):
            for j in range(G // 16):
                sv = srcbuf[pl.ds(b * G + j * 16, 16)] * 2
                dv = ldstbuf[pl.ds(b * G + j * 16, 16)] * 2
                for k in range(6):
                    gidx[k, pl.ds(j * 16, 16)] = sv + ((k // 2) * XB
                                                       + (k % 2))
                    sidx[k, pl.ds(j * 16, 16)] = dv + ((k // 2) * ACC_B
                                                       + (k % 2))
            cps = [pltpu.async_copy(x_hbm.at[gidx.at[k]], rows_v.at[k], sem)
                   for k in range(6)]
            scps = []
            for k in range(6):
                cps[k].wait()
                scps.append(pltpu.async_copy(
                    rows_v.at[k], acc_sh.at[sidx.at[k]], zsem, add=True))
            for cp in scps:
                cp.wait()
            return carry
        lax.fori_loop(0, nb, _gs, 0)
        plsc.subcore_barrier()

        wb_prev = []
        for k in range(3):
            wrow = 2 * sid * WB
            wb_prev.append(pltpu.async_copy(
                acc_sh.at[pl.ds(k * ACC_B + wrow, 2 * WB)],
                out_hbm.at[pl.ds(k * OB + 2 * lo + wrow, 2 * WB)], bsem))
    for cp in wb_prev:
        cp.wait()


@functools.lru_cache(maxsize=1)
def _get_seg_sum():
    return pl.kernel(
        _sc_body,
        out_type=jax.ShapeDtypeStruct((3 * OB, 128), jnp.float32),
        mesh=plsc.VectorSubcoreMesh(core_axis_name="c", subcore_axis_name="s",
                                    num_cores=NC, num_subcores=NS),
        compiler_params=pltpu.CompilerParams(needs_layout_passes=False),
        scratch_types=[
            pltpu.VMEM((WE,), jnp.int32),
            pltpu.VMEM((WE,), jnp.int32),
            pltpu.VMEM((WE,), jnp.int32),
            pltpu.VMEM((WE,), jnp.int32),
            pltpu.VMEM((BUF + G,), jnp.int32),
            pltpu.VMEM((BUF + G,), jnp.int32),
            pltpu.VMEM((6, G), jnp.int32),
            pltpu.VMEM((6, G), jnp.int32),
            pltpu.VMEM((6, G, 128), jnp.float32),
            pltpu.VMEM((ZROWS, 128), jnp.float32),
            pltpu.VMEM_SHARED((3 * ACC_B, 128), jnp.float32),
            pltpu.SemaphoreType.DMA,
            pltpu.SemaphoreType.DMA,
            pltpu.SemaphoreType.DMA,
            pltpu.SemaphoreType.DMA,
        ],
    )


BN = 400


def _tc_body(x_ref, p_ref, wid_ref, wpool_ref, wdir_ref, bias_ref, out_ref):
    xb = x_ref[...]
    x2 = xb.reshape(3 * BN, C)
    p2 = p_ref[...].reshape(3 * BN, C)
    emb = (jnp.dot(x2.astype(jnp.bfloat16), wid_ref[...],
                   preferred_element_type=jnp.float32)
           + jnp.dot(p2.astype(jnp.bfloat16), wpool_ref[...],
                     preferred_element_type=jnp.float32)
           + bias_ref[...])
    d = jnp.dot(emb.astype(jnp.bfloat16), wdir_ref[...],
                preferred_element_type=jnp.float32)
    e3 = emb.reshape(3, BN, C)
    d3 = d.reshape(3, BN, C)
    dot = jnp.sum(e3 * d3, axis=0, keepdims=True)
    dn = jnp.sum(d3 * d3, axis=0, keepdims=True)
    coef = jnp.where(dot < 0.0, dot / (dn + EPS), 0.0)
    out_ref[...] = e3 - coef * d3 + xb


_tc_call = pl.pallas_call(
    _tc_body,
    grid=(N // BN,),
    in_specs=[
        pl.BlockSpec((3, BN, C), lambda i: (0, i, 0)),
        pl.BlockSpec((3, BN, C), lambda i: (0, i, 0)),
        pl.BlockSpec((C, C), lambda i: (0, 0)),
        pl.BlockSpec((C, C), lambda i: (0, 0)),
        pl.BlockSpec((C, C), lambda i: (0, 0)),
        pl.BlockSpec((1, C), lambda i: (0, 0)),
    ],
    out_specs=pl.BlockSpec((3, BN, C), lambda i: (0, i, 0)),
    out_shape=jax.ShapeDtypeStruct((3, N, C), jnp.float32),
)


def kernel(x, edges, W_id, b_id, W_pool, b_pool, W_dir):
    xt = jnp.transpose(x, (1, 0, 2))
    x2d = xt.reshape(3 * XB, 128)
    src = edges[0]
    dst = edges[1]
    pooled = _get_seg_sum()(x2d, src, dst)
    pooled_t = pooled.reshape(3, NPAD, C)
    bias = (b_id + b_pool).reshape(1, C)
    out_t = _tc_call(xt, pooled_t,
                     W_id.T.astype(jnp.bfloat16),
                     W_pool.T.astype(jnp.bfloat16),
                     W_dir.T.astype(jnp.bfloat16), bias)
    out = jnp.transpose(out_t, (1, 0, 2))
    return (out, edges)
